# baseline (speedup 1.0000x reference)
import functools

import jax
import jax.numpy as jnp
from jax import lax
from jax.experimental import pallas as pl
from jax.experimental.pallas import tpu as pltpu
from jax.experimental.pallas import tpu_sc as plsc

MAXLEN = 200
D = 32
B = 4096
S = 200
NW = 32
BBLK = B // NW
LANES = 16

_mesh = plsc.VectorSubcoreMesh(core_axis_name="c", subcore_axis_name="s")


@functools.partial(
    pl.kernel,
    mesh=_mesh,
    out_type=jax.ShapeDtypeStruct((S, NW, BBLK, D), jnp.float32),
    compiler_params=pltpu.CompilerParams(
        use_tc_tiling_on_sc=False, needs_layout_passes=False,
        disable_bounds_checks=True),
    scratch_types=[
        pltpu.VMEM((S, BBLK), jnp.int32),
        pltpu.VMEM((MAXLEN, D), jnp.float32),
        pltpu.VMEM((BBLK, D), jnp.float32),
        pltpu.VMEM((BBLK, D), jnp.float32),
        pltpu.SemaphoreType.DMA,
        pltpu.SemaphoreType.DMA,
    ],
)
def _embed(xt_hbm, tok_hbm, pos_hbm, out_hbm, idx_v, pos_v, rows0, rows1, sem0, sem1):
    wid = lax.axis_index("s") * 2 + lax.axis_index("c")
    b0 = wid * BBLK

    pltpu.sync_copy(xt_hbm.at[:, pl.ds(b0, BBLK)], idx_v)
    pltpu.sync_copy(pos_hbm, pos_v)

    rows = (rows0, rows1)
    sems = (sem0, sem1)

    def gather_start(s, b):
        pltpu.async_copy(tok_hbm.at[idx_v.at[s]], rows[b], sems[b])

    def gather_wait(s, b):
        pltpu.make_async_copy(tok_hbm.at[idx_v.at[s]], rows[b], sems[b]).wait()

    gather_start(0, 0)
    gather_start(1, 1)

    def chunk_body(ss, carry):
        for b in range(2):
            s = 2 * ss + b
            gather_wait(s, b)
            p0 = pos_v[s, pl.ds(0, LANES)]
            p1 = pos_v[s, pl.ds(LANES, LANES)]

            def row_body(i, carry2, _b=b, _p0=p0, _p1=p1):
                rows[_b][i, pl.ds(0, LANES)] = rows[_b][i, pl.ds(0, LANES)] + _p0
                rows[_b][i, pl.ds(LANES, LANES)] = rows[_b][i, pl.ds(LANES, LANES)] + _p1
                return carry2

            lax.fori_loop(0, BBLK, row_body, 0, unroll=4)
            pltpu.sync_copy(rows[b], out_hbm.at[s, wid])

            @pl.when(s + 2 < S)
            def _(_s=s, _b=b):
                gather_start(_s + 2, _b)

        return carry

    lax.fori_loop(0, S // 2, chunk_body, 0)


def kernel(x, token_table, pos_table):
    xt = x.astype(jnp.int32).T
    out = _embed(xt, token_table, pos_table)
    return out.transpose(1, 2, 0, 3).reshape(B, S, D)
